# baseline (speedup 1.0000x reference)
import jax
import jax.numpy as jnp
from jax import lax
from jax.experimental import pallas as pl
from jax.experimental.pallas import tpu as pltpu
from jax.experimental.pallas import tpu_sc as plsc

VOCAB_ = 1000000
D_ = 64
BATCH_ = 16384
HIST_ = 50

NC_ = 2
NS_ = 16
NW_ = NC_ * NS_

CG_ = 4
B_PER_W_ = BATCH_ // NW_
N_ = B_PER_W_ // CG_
NBUF_ = 4
P_ = 2
GROUPS_ = N_ // NBUF_
ROWS_PER_B_ = HIST_ * D_ // 128


def _emb_kernel(table_hbm, idx_hbm, out_hbm, idx_v, rows_v, *sems):
    gsems = sems[:NBUF_]
    wsems = sems[NBUF_:]
    wid = lax.axis_index("s") * NC_ + lax.axis_index("c")
    base = wid * B_PER_W_

    pltpu.sync_copy(idx_hbm.at[pl.ds(base, B_PER_W_)], idx_v)

    def out_slice(t):
        r0 = pl.multiple_of((base + t * CG_) * HIST_, CG_ * HIST_)
        return out_hbm.at[pl.ds(r0, CG_ * HIST_)]

    def chunk(b):
        return rows_v.at[pl.ds(b * CG_ * HIST_, CG_ * HIST_)]

    def fire_gathers(t, b):
        for j in range(CG_):
            pltpu.async_copy(
                table_hbm.at[idx_v.at[t * CG_ + j]],
                rows_v.at[pl.ds((b * CG_ + j) * HIST_, HIST_)],
                gsems[b],
            )

    def wait_gathers(t, b):
        pltpu.make_async_copy(out_slice(t), chunk(b), gsems[b]).wait()

    def fire_write(t, b):
        pltpu.async_copy(chunk(b), out_slice(t), wsems[b])

    def wait_write(t, b):
        pltpu.make_async_copy(chunk(b), out_slice(t), wsems[b]).wait()

    def turn(t, b, bf, fire, drain_w):
        if fire:
            if drain_w:
                wait_write(t + P_ - NBUF_, bf)
            fire_gathers(t + P_, bf)
        wait_gathers(t, b)
        fire_write(t, b)

    fire_gathers(0, 0)
    fire_gathers(1, 1)

    turn(0, 0, 2, True, False)
    turn(1, 1, 3, True, False)
    turn(2, 2, 0, True, True)
    turn(3, 3, 1, True, True)

    def body(gg, _):
        t0 = gg * NBUF_
        for b in range(NBUF_):
            turn(t0 + b, b, (b + P_) % NBUF_, True, True)
        return ()

    lax.fori_loop(1, GROUPS_ - 1, body, (), unroll=False)

    tl = N_ - NBUF_
    turn(tl + 0, 0, 2, True, True)
    turn(tl + 1, 1, 3, True, True)
    turn(tl + 2, 2, 0, False, False)
    turn(tl + 3, 3, 1, False, False)

    for b in range(NBUF_):
        wait_write(N_ - NBUF_ + b, b)


def _identity_rootcap(x):
    return pl.pallas_call(
        lambda x_ref, o_ref: None,
        out_shape=jax.ShapeDtypeStruct(x.shape, x.dtype),
        in_specs=[pl.BlockSpec(memory_space=pl.ANY)],
        out_specs=pl.BlockSpec(memory_space=pl.ANY),
        input_output_aliases={0: 0},
    )(x)


@jax.jit
def kernel(token_ids, hidden):
    idx_2d = token_ids.astype(jnp.int32)

    mesh = plsc.VectorSubcoreMesh(core_axis_name="c", subcore_axis_name="s")
    run = pl.kernel(
        _emb_kernel,
        out_type=jax.ShapeDtypeStruct((BATCH_ * HIST_, D_), jnp.float32),
        mesh=mesh,
        scratch_types=[
            pltpu.VMEM((B_PER_W_, HIST_), jnp.int32),
            pltpu.VMEM((NBUF_ * CG_ * HIST_, D_), jnp.float32),
        ]
        + [pltpu.SemaphoreType.DMA] * (2 * NBUF_),
        compiler_params=pltpu.CompilerParams(use_tc_tiling_on_sc=False),
    )
    out = run(hidden, idx_2d).reshape(BATCH_, HIST_, D_)
    return _identity_rootcap(out)


# speedup vs baseline: 1.1073x; 1.1073x over previous
import jax
import jax.numpy as jnp
from jax import lax
from jax.experimental import pallas as pl
from jax.experimental.pallas import tpu as pltpu
from jax.experimental.pallas import tpu_sc as plsc

VOCAB_ = 1000000
D_ = 64
B_TOTAL_ = 16384 * 50

NC_ = 2
NS_ = 16
NW_ = NC_ * NS_

SUB_ = 128
CHUNK_ = 256
SPC_ = CHUNK_ // SUB_
B_PER_W_ = B_TOTAL_ // NW_
N_ = B_PER_W_ // CHUNK_
NBUF_ = 4
P_ = 2
GROUPS_ = N_ // NBUF_
IROWS_ = B_PER_W_ // SUB_


def _emb_kernel(table_hbm, idx_hbm, out_hbm, idx_v, rows_v, *sems):
    gsems = sems[:NBUF_]
    wsems = sems[NBUF_:]
    wid = lax.axis_index("s") * NC_ + lax.axis_index("c")
    base = wid * B_PER_W_

    pltpu.sync_copy(idx_hbm.at[pl.ds(wid * IROWS_, IROWS_)], idx_v)

    def out_slice(t):
        r0 = pl.multiple_of(base + t * CHUNK_, CHUNK_)
        return out_hbm.at[pl.ds(r0, CHUNK_)]

    def chunk(b):
        return rows_v.at[pl.ds(b * CHUNK_, CHUNK_)]

    def fire_gathers(t, b):
        for j in range(SPC_):
            pltpu.async_copy(
                table_hbm.at[idx_v.at[t * SPC_ + j]],
                rows_v.at[pl.ds(b * CHUNK_ + j * SUB_, SUB_)],
                gsems[b],
            )

    def wait_gathers(t, b):
        pltpu.make_async_copy(out_slice(t), chunk(b), gsems[b]).wait()

    def fire_write(t, b):
        pltpu.async_copy(chunk(b), out_slice(t), wsems[b])

    def wait_write(t, b):
        pltpu.make_async_copy(chunk(b), out_slice(t), wsems[b]).wait()

    def turn(t, b, bf, fire, drain_w):
        if fire:
            if drain_w:
                wait_write(t + P_ - NBUF_, bf)
            fire_gathers(t + P_, bf)
        wait_gathers(t, b)
        fire_write(t, b)

    fire_gathers(0, 0)
    fire_gathers(1, 1)

    turn(0, 0, 2, True, False)
    turn(1, 1, 3, True, False)
    turn(2, 2, 0, True, True)
    turn(3, 3, 1, True, True)

    def body(gg, _):
        t0 = gg * NBUF_
        for b in range(NBUF_):
            turn(t0 + b, b, (b + P_) % NBUF_, True, True)
        return ()

    lax.fori_loop(1, GROUPS_ - 1, body, (), unroll=False)

    tl = N_ - NBUF_
    turn(tl + 0, 0, 2, True, True)
    turn(tl + 1, 1, 3, True, True)
    turn(tl + 2, 2, 0, False, False)
    turn(tl + 3, 3, 1, False, False)

    for b in range(NBUF_):
        wait_write(N_ - NBUF_ + b, b)


@jax.jit
def kernel(token_ids, hidden):
    idx_2d = token_ids.reshape(B_TOTAL_ // SUB_, SUB_).astype(jnp.int32)

    mesh = plsc.VectorSubcoreMesh(core_axis_name="c", subcore_axis_name="s")
    run = pl.kernel(
        _emb_kernel,
        out_type=jax.ShapeDtypeStruct((B_TOTAL_, D_), jnp.float32),
        mesh=mesh,
        scratch_types=[
            pltpu.VMEM((IROWS_, SUB_), jnp.int32),
            pltpu.VMEM((NBUF_ * CHUNK_, D_), jnp.float32),
        ]
        + [pltpu.SemaphoreType.DMA] * (2 * NBUF_),
        compiler_params=pltpu.CompilerParams(use_tc_tiling_on_sc=False),
    )
    out = run(hidden, idx_2d)
    return out.reshape(token_ids.shape + (D_,))
